# Initial kernel scaffold; baseline (speedup 1.0000x reference)
import jax
import jax.numpy as jnp
from jax.experimental import pallas as pl


def kernel(x, edge_index, params):
    raise NotImplementedError("write your pallas kernel here")


# speedup vs baseline: 3.3829x; 3.3829x over previous
import functools

import jax
import jax.numpy as jnp
from jax import lax
from jax.experimental import pallas as pl
from jax.experimental.pallas import tpu as pltpu
from jax.experimental.pallas import tpu_sc as plsc

_NC = 2
_NS = 16
_FC = 4


def _tc_mlp2_chunked(xx, w1, b1, w2, b2, npad):
    n, kin = xx.shape
    kh = w1.shape[1]
    br = 400

    def body(x_ref, w1_ref, b1_ref, w2_ref, b2_ref, o_ref):
        g = jnp.dot(x_ref[...], w1_ref[...], preferred_element_type=jnp.float32)
        g = jnp.maximum(g + b1_ref[...], 0.0)
        f = jnp.dot(g, w2_ref[...], preferred_element_type=jnp.float32) + b2_ref[...]
        for c in range(_FC):
            o_ref[c] = f[:, c * 128:(c + 1) * 128]

    return pl.pallas_call(
        body,
        grid=(n // br,),
        in_specs=[
            pl.BlockSpec((br, kin), lambda i: (i, 0)),
            pl.BlockSpec((kin, kh), lambda i: (0, 0)),
            pl.BlockSpec((1, kh), lambda i: (0, 0)),
            pl.BlockSpec((kh, kh), lambda i: (0, 0)),
            pl.BlockSpec((1, kh), lambda i: (0, 0)),
        ],
        out_specs=pl.BlockSpec((_FC, br, 128), lambda i: (0, i, 0)),
        out_shape=jax.ShapeDtypeStruct((_FC, npad, 128), jnp.float32),
    )(xx, w1, b1.reshape(1, -1), w2, b2.reshape(1, -1))


def _tc_out_mlp(fc, ac, p, n):
    br = 400
    eps = p['eps'].reshape(1, 1)
    bias = (p['o_b2'] + p['s_b']).reshape(1, -1)

    def body(f_ref, a_ref, eps_ref, w1_ref, b1_ref, w2_ref, sw_ref, b_ref,
             g_ref, bb_ref, o_ref):
        f = jnp.concatenate([f_ref[c] for c in range(_FC)], axis=-1)
        a = jnp.concatenate([a_ref[c] for c in range(_FC)], axis=-1)
        t = (1.0 + eps_ref[0, 0]) * f + a
        u = jnp.dot(t, w1_ref[...], preferred_element_type=jnp.float32)
        u = jnp.maximum(u + b1_ref[...], 0.0)
        v = (jnp.dot(u, w2_ref[...], preferred_element_type=jnp.float32)
             + jnp.dot(a, sw_ref[...], preferred_element_type=jnp.float32)
             + b_ref[...])
        mu = jnp.mean(v, axis=-1, keepdims=True)
        var = jnp.mean((v - mu) ** 2, axis=-1, keepdims=True)
        o_ref[...] = (v - mu) / jnp.sqrt(var + 1e-5) * g_ref[...] + bb_ref[...]

    d = fc.shape[0] * fc.shape[2]
    return pl.pallas_call(
        body,
        grid=(n // br,),
        in_specs=[
            pl.BlockSpec((_FC, br, 128), lambda i: (0, i, 0)),
            pl.BlockSpec((_FC, br, 128), lambda i: (0, i, 0)),
            pl.BlockSpec((1, 1), lambda i: (0, 0)),
            pl.BlockSpec((d, d), lambda i: (0, 0)),
            pl.BlockSpec((1, d), lambda i: (0, 0)),
            pl.BlockSpec((d, d), lambda i: (0, 0)),
            pl.BlockSpec((d, d), lambda i: (0, 0)),
            pl.BlockSpec((1, d), lambda i: (0, 0)),
            pl.BlockSpec((1, d), lambda i: (0, 0)),
            pl.BlockSpec((1, d), lambda i: (0, 0)),
        ],
        out_specs=pl.BlockSpec((br, d), lambda i: (i, 0)),
        out_shape=jax.ShapeDtypeStruct((n, d), jnp.float32),
    )(fc, ac, eps, p['o_W1'], p['o_b1'].reshape(1, -1), p['o_W2'], p['s_W'],
      bias, p['ln_g'].reshape(1, -1), p['ln_b'].reshape(1, -1))


def _tc_dual_mm(h, wt, wb):
    n, d = h.shape
    br = 400

    def body(h_ref, wt_ref, wb_ref, o_ref):
        o_ref[0] = jnp.dot(h_ref[...], wt_ref[...], preferred_element_type=jnp.float32)
        o_ref[1] = jnp.dot(h_ref[...], wb_ref[...], preferred_element_type=jnp.float32)

    return pl.pallas_call(
        body,
        grid=(n // br,),
        in_specs=[
            pl.BlockSpec((br, d), lambda i: (i, 0)),
            pl.BlockSpec((d, d), lambda i: (0, 0)),
            pl.BlockSpec((d, d), lambda i: (0, 0)),
        ],
        out_specs=pl.BlockSpec((2, br, d), lambda i: (0, i, 0)),
        out_shape=jax.ShapeDtypeStruct((2, n, d), jnp.float32),
    )(h, wt, wb)


def _tc_edge_mlp(ps, qd, b1, w2, b2):
    e, d = ps.shape
    ko = w2.shape[1]
    br = 400

    def body(p_ref, q_ref, b1_ref, w2_ref, b2_ref, o_ref):
        r = jnp.maximum(p_ref[...] + q_ref[...] + b1_ref[...], 0.0)
        o_ref[...] = jnp.dot(r, w2_ref[...], preferred_element_type=jnp.float32) + b2_ref[...]

    return pl.pallas_call(
        body,
        grid=(e // br,),
        in_specs=[
            pl.BlockSpec((br, d), lambda i: (i, 0)),
            pl.BlockSpec((br, d), lambda i: (i, 0)),
            pl.BlockSpec((1, d), lambda i: (0, 0)),
            pl.BlockSpec((d, ko), lambda i: (0, 0)),
            pl.BlockSpec((1, ko), lambda i: (0, 0)),
        ],
        out_specs=pl.BlockSpec((br, ko), lambda i: (i, 0)),
        out_shape=jax.ShapeDtypeStruct((e, ko), jnp.float32),
    )(ps, qd, b1.reshape(1, -1), w2, b2.reshape(1, -1))


def _sc_aggregate(fc, srcoff, dst, npad):
    e = dst.shape[0]
    blk = 80
    ept = e // _NS
    nb = ept // blk
    rpt = npad // _NS
    mesh = plsc.VectorSubcoreMesh(core_axis_name="c", subcore_axis_name="s")

    @functools.partial(
        pl.kernel,
        mesh=mesh,
        out_type=jax.ShapeDtypeStruct((_FC * npad, 128), jnp.float32),
        scratch_types=[
            pltpu.VMEM_SHARED((npad, 128), jnp.float32),
            pltpu.VMEM((blk,), jnp.int32),
            pltpu.VMEM((blk,), jnp.int32),
            pltpu.VMEM((blk, 128), jnp.float32),
            pltpu.SemaphoreType.DMA,
        ],
    )
    def k(fc_hbm, srcoff_hbm, dst_hbm, agg_hbm, shared, si_v, di_v, rows_v, sem):
        cid = lax.axis_index("c")
        sid = lax.axis_index("s")
        e0 = sid * ept
        for c2 in range(_FC // _NC):
            chunk = cid * (_FC // _NC) + c2
            crow = chunk * npad
            pltpu.sync_copy(fc_hbm.at[pl.ds(crow + sid * rpt, rpt)],
                            shared.at[pl.ds(sid * rpt, rpt)])
            plsc.subcore_barrier()

            def step(b, carry):
                off = e0 + b * blk
                pltpu.sync_copy(srcoff_hbm.at[pl.ds(chunk * e + off, blk)], si_v)
                pltpu.sync_copy(dst_hbm.at[pl.ds(off, blk)], di_v)
                pltpu.async_copy(fc_hbm.at[si_v], rows_v, sem).wait()
                pltpu.sync_copy(rows_v, shared.at[di_v], add=True)
                return carry

            lax.fori_loop(0, nb, step, 0)
            plsc.subcore_barrier()
            pltpu.sync_copy(shared.at[pl.ds(sid * rpt, rpt)],
                            agg_hbm.at[pl.ds(crow + sid * rpt, rpt)])
            plsc.subcore_barrier()

    return k(fc, srcoff, dst)


def _sc_edge_gather(pqc, si, di):
    e = si.shape[0]
    d = pqc.shape[1]
    blk = 40
    perw = e // (_NC * _NS)
    nb = perw // blk
    mesh = plsc.VectorSubcoreMesh(core_axis_name="c", subcore_axis_name="s")

    @functools.partial(
        pl.kernel,
        mesh=mesh,
        out_type=(jax.ShapeDtypeStruct((e, d), jnp.float32),
                  jax.ShapeDtypeStruct((e, d), jnp.float32)),
        scratch_types=[
            pltpu.VMEM((blk,), jnp.int32),
            pltpu.VMEM((blk,), jnp.int32),
            pltpu.VMEM((blk, d), jnp.float32),
            pltpu.VMEM((blk, d), jnp.float32),
            pltpu.SemaphoreType.DMA,
            pltpu.SemaphoreType.DMA,
        ],
    )
    def k(pq_hbm, si_hbm, di_hbm, ps_hbm, qd_hbm, si_v, di_v, pr_v, qr_v, s1, s2):
        wid = lax.axis_index("s") * _NC + lax.axis_index("c")
        base = wid * perw

        def step(b, carry):
            off = base + b * blk
            pltpu.sync_copy(si_hbm.at[pl.ds(off, blk)], si_v)
            pltpu.sync_copy(di_hbm.at[pl.ds(off, blk)], di_v)
            cp1 = pltpu.async_copy(pq_hbm.at[si_v], pr_v, s1)
            cp2 = pltpu.async_copy(pq_hbm.at[di_v], qr_v, s2)
            cp1.wait()
            cp2.wait()
            pltpu.sync_copy(pr_v, ps_hbm.at[pl.ds(off, blk)])
            pltpu.sync_copy(qr_v, qd_hbm.at[pl.ds(off, blk)])
            return carry

        lax.fori_loop(0, nb, step, 0)

    return k(pqc, si, di)


def kernel(x, edge_index, params):
    n = x.shape[0]
    npad = ((n + 127) // 128) * 128
    src = edge_index[0]
    dst = edge_index[1]
    srcoff = (src[None, :]
              + (jnp.arange(_FC, dtype=jnp.int32) * npad)[:, None]).reshape(-1)

    h = x
    for p in params['layers']:
        fc = _tc_mlp2_chunked(h, p['f_W1'], p['f_b1'], p['f_W2'], p['f_b2'], npad)
        aggc = _sc_aggregate(fc.reshape(_FC * npad, 128), srcoff, dst, npad)
        h = _tc_out_mlp(fc, aggc.reshape(_FC, npad, 128), p, n)

    w1 = params['lp_W1']
    dh = h.shape[1]
    pq = _tc_dual_mm(h, w1[:dh], w1[dh:])
    ps, qd = _sc_edge_gather(pq.reshape(2 * n, dh), src, dst + n)
    return _tc_edge_mlp(ps, qd, params['lp_b1'], params['lp_W2'], params['lp_b2'])


# speedup vs baseline: 4.7253x; 1.3968x over previous
import functools

import jax
import jax.numpy as jnp
from jax import lax
from jax.experimental import pallas as pl
from jax.experimental.pallas import tpu as pltpu
from jax.experimental.pallas import tpu_sc as plsc

_NC = 2
_NS = 16
_FC = 4


def _tc_mlp2_chunked(xx, w1, b1, w2, b2, npad):
    n, kin = xx.shape
    kh = w1.shape[1]
    br = 400

    def body(x_ref, w1_ref, b1_ref, w2_ref, b2_ref, o_ref):
        g = jnp.dot(x_ref[...], w1_ref[...], preferred_element_type=jnp.float32)
        g = jnp.maximum(g + b1_ref[...], 0.0)
        f = jnp.dot(g, w2_ref[...], preferred_element_type=jnp.float32) + b2_ref[...]
        for c in range(_FC):
            o_ref[c] = f[:, c * 128:(c + 1) * 128]

    return pl.pallas_call(
        body,
        grid=(n // br,),
        in_specs=[
            pl.BlockSpec((br, kin), lambda i: (i, 0)),
            pl.BlockSpec((kin, kh), lambda i: (0, 0)),
            pl.BlockSpec((1, kh), lambda i: (0, 0)),
            pl.BlockSpec((kh, kh), lambda i: (0, 0)),
            pl.BlockSpec((1, kh), lambda i: (0, 0)),
        ],
        out_specs=pl.BlockSpec((_FC, br, 128), lambda i: (0, i, 0)),
        out_shape=jax.ShapeDtypeStruct((_FC, npad, 128), jnp.float32),
    )(xx, w1, b1.reshape(1, -1), w2, b2.reshape(1, -1))


def _tc_out_mlp(fc, ac, p, n):
    br = 400
    eps = p['eps'].reshape(1, 1)
    bias = (p['o_b2'] + p['s_b']).reshape(1, -1)

    def body(f_ref, a_ref, eps_ref, w1_ref, b1_ref, w2_ref, sw_ref, b_ref,
             g_ref, bb_ref, o_ref):
        f = jnp.concatenate([f_ref[c] for c in range(_FC)], axis=-1)
        a = jnp.concatenate([a_ref[c] for c in range(_FC)], axis=-1)
        t = (1.0 + eps_ref[0, 0]) * f + a
        u = jnp.dot(t, w1_ref[...], preferred_element_type=jnp.float32)
        u = jnp.maximum(u + b1_ref[...], 0.0)
        v = (jnp.dot(u, w2_ref[...], preferred_element_type=jnp.float32)
             + jnp.dot(a, sw_ref[...], preferred_element_type=jnp.float32)
             + b_ref[...])
        mu = jnp.mean(v, axis=-1, keepdims=True)
        var = jnp.mean((v - mu) ** 2, axis=-1, keepdims=True)
        o_ref[...] = (v - mu) / jnp.sqrt(var + 1e-5) * g_ref[...] + bb_ref[...]

    d = fc.shape[0] * fc.shape[2]
    return pl.pallas_call(
        body,
        grid=(n // br,),
        in_specs=[
            pl.BlockSpec((_FC, br, 128), lambda i: (0, i, 0)),
            pl.BlockSpec((_FC, br, 128), lambda i: (0, i, 0)),
            pl.BlockSpec((1, 1), lambda i: (0, 0)),
            pl.BlockSpec((d, d), lambda i: (0, 0)),
            pl.BlockSpec((1, d), lambda i: (0, 0)),
            pl.BlockSpec((d, d), lambda i: (0, 0)),
            pl.BlockSpec((d, d), lambda i: (0, 0)),
            pl.BlockSpec((1, d), lambda i: (0, 0)),
            pl.BlockSpec((1, d), lambda i: (0, 0)),
            pl.BlockSpec((1, d), lambda i: (0, 0)),
        ],
        out_specs=pl.BlockSpec((br, d), lambda i: (i, 0)),
        out_shape=jax.ShapeDtypeStruct((n, d), jnp.float32),
    )(fc, ac, eps, p['o_W1'], p['o_b1'].reshape(1, -1), p['o_W2'], p['s_W'],
      bias, p['ln_g'].reshape(1, -1), p['ln_b'].reshape(1, -1))


def _tc_dual_mm(h, wt, wb):
    n, d = h.shape
    br = 400

    def body(h_ref, wt_ref, wb_ref, o_ref):
        o_ref[0] = jnp.dot(h_ref[...], wt_ref[...], preferred_element_type=jnp.float32)
        o_ref[1] = jnp.dot(h_ref[...], wb_ref[...], preferred_element_type=jnp.float32)

    return pl.pallas_call(
        body,
        grid=(n // br,),
        in_specs=[
            pl.BlockSpec((br, d), lambda i: (i, 0)),
            pl.BlockSpec((d, d), lambda i: (0, 0)),
            pl.BlockSpec((d, d), lambda i: (0, 0)),
        ],
        out_specs=pl.BlockSpec((2, br, d), lambda i: (0, i, 0)),
        out_shape=jax.ShapeDtypeStruct((2, n, d), jnp.float32),
    )(h, wt, wb)


def _tc_edge_mlp(ps, qd, b1, w2, b2):
    e, d = ps.shape
    ko = w2.shape[1]
    br = 400

    def body(p_ref, q_ref, b1_ref, w2_ref, b2_ref, o_ref):
        r = jnp.maximum(p_ref[...] + q_ref[...] + b1_ref[...], 0.0)
        o_ref[...] = jnp.dot(r, w2_ref[...], preferred_element_type=jnp.float32) + b2_ref[...]

    return pl.pallas_call(
        body,
        grid=(e // br,),
        in_specs=[
            pl.BlockSpec((br, d), lambda i: (i, 0)),
            pl.BlockSpec((br, d), lambda i: (i, 0)),
            pl.BlockSpec((1, d), lambda i: (0, 0)),
            pl.BlockSpec((d, ko), lambda i: (0, 0)),
            pl.BlockSpec((1, ko), lambda i: (0, 0)),
        ],
        out_specs=pl.BlockSpec((br, ko), lambda i: (i, 0)),
        out_shape=jax.ShapeDtypeStruct((e, ko), jnp.float32),
    )(ps, qd, b1.reshape(1, -1), w2, b2.reshape(1, -1))


def _sc_aggregate(fc, srcoff, dst, npad):
    e = dst.shape[0]
    blk = 80
    ept = e // _NS
    nb = ept // blk
    rpt = npad // _NS
    mesh = plsc.VectorSubcoreMesh(core_axis_name="c", subcore_axis_name="s")

    @functools.partial(
        pl.kernel,
        mesh=mesh,
        out_type=jax.ShapeDtypeStruct((_FC * npad, 128), jnp.float32),
        scratch_types=[
            pltpu.VMEM_SHARED((npad, 128), jnp.float32),
            pltpu.VMEM((blk,), jnp.int32),
            pltpu.VMEM((blk,), jnp.int32),
            pltpu.VMEM((blk,), jnp.int32),
            pltpu.VMEM((blk,), jnp.int32),
            pltpu.VMEM((blk, 128), jnp.float32),
            pltpu.VMEM((blk, 128), jnp.float32),
            pltpu.SemaphoreType.DMA,
            pltpu.SemaphoreType.DMA,
        ],
    )
    def k(fc_hbm, srcoff_hbm, dst_hbm, agg_hbm, shared,
          si0, di0, si1, di1, rows0, rows1, sem0, sem1):
        cid = lax.axis_index("c")
        sid = lax.axis_index("s")
        e0 = sid * ept
        bufs = ((si0, di0, rows0, sem0), (si1, di1, rows1, sem1))

        def start(b, j):
            si, di, rows, sem = bufs[j]
            off = e0 + b * blk
            pltpu.sync_copy(srcoff_hbm.at[pl.ds(chunk * e + off, blk)], si)
            pltpu.sync_copy(dst_hbm.at[pl.ds(off, blk)], di)
            pltpu.async_copy(fc_hbm.at[si], rows, sem)

        def finish(j):
            si, di, rows, sem = bufs[j]
            pltpu.make_async_copy(fc_hbm.at[si], rows, sem).wait()
            pltpu.sync_copy(rows, shared.at[di], add=True)

        for c2 in range(_FC // _NC):
            chunk = cid * (_FC // _NC) + c2
            crow = chunk * npad
            pltpu.sync_copy(fc_hbm.at[pl.ds(crow + sid * rpt, rpt)],
                            shared.at[pl.ds(sid * rpt, rpt)])
            plsc.subcore_barrier()

            start(0, 0)

            def step(g, carry):
                b0 = 2 * g

                @pl.when(b0 + 1 < nb)
                def _():
                    start(b0 + 1, 1)

                finish(0)

                @pl.when(b0 + 2 < nb)
                def _():
                    start(b0 + 2, 0)

                @pl.when(b0 + 1 < nb)
                def _():
                    finish(1)

                return carry

            lax.fori_loop(0, (nb + 1) // 2, step, 0)
            plsc.subcore_barrier()
            pltpu.sync_copy(shared.at[pl.ds(sid * rpt, rpt)],
                            agg_hbm.at[pl.ds(crow + sid * rpt, rpt)])
            plsc.subcore_barrier()

    return k(fc, srcoff, dst)


def _sc_edge_gather(pqc, si, di):
    e = si.shape[0]
    d = pqc.shape[1]
    blk = 40
    perw = e // (_NC * _NS)
    nb = perw // blk
    mesh = plsc.VectorSubcoreMesh(core_axis_name="c", subcore_axis_name="s")

    @functools.partial(
        pl.kernel,
        mesh=mesh,
        out_type=(jax.ShapeDtypeStruct((e, d), jnp.float32),
                  jax.ShapeDtypeStruct((e, d), jnp.float32)),
        scratch_types=[
            pltpu.VMEM((blk,), jnp.int32),
            pltpu.VMEM((blk,), jnp.int32),
            pltpu.VMEM((blk,), jnp.int32),
            pltpu.VMEM((blk,), jnp.int32),
            pltpu.VMEM((blk, d), jnp.float32),
            pltpu.VMEM((blk, d), jnp.float32),
            pltpu.VMEM((blk, d), jnp.float32),
            pltpu.VMEM((blk, d), jnp.float32),
            pltpu.SemaphoreType.DMA,
            pltpu.SemaphoreType.DMA,
        ],
    )
    def k(pq_hbm, si_hbm, di_hbm, ps_hbm, qd_hbm,
          si0, di0, si1, di1, pr0, qr0, pr1, qr1, s0, s1):
        wid = lax.axis_index("s") * _NC + lax.axis_index("c")
        base = wid * perw
        bufs = ((si0, di0, pr0, qr0, s0), (si1, di1, pr1, qr1, s1))

        def start(b, j):
            si, di, pr, qr, sm = bufs[j]
            off = base + b * blk
            pltpu.sync_copy(si_hbm.at[pl.ds(off, blk)], si)
            pltpu.sync_copy(di_hbm.at[pl.ds(off, blk)], di)
            pltpu.async_copy(pq_hbm.at[si], pr, sm)
            pltpu.async_copy(pq_hbm.at[di], qr, sm)

        def finish(b, j):
            si, di, pr, qr, sm = bufs[j]
            off = base + b * blk
            pltpu.make_async_copy(pq_hbm.at[si], pr, sm).wait()
            pltpu.make_async_copy(pq_hbm.at[di], qr, sm).wait()
            pltpu.sync_copy(pr, ps_hbm.at[pl.ds(off, blk)])
            pltpu.sync_copy(qr, qd_hbm.at[pl.ds(off, blk)])

        start(0, 0)

        def step(g, carry):
            b0 = 2 * g

            @pl.when(b0 + 1 < nb)
            def _():
                start(b0 + 1, 1)

            finish(b0, 0)

            @pl.when(b0 + 2 < nb)
            def _():
                start(b0 + 2, 0)

            @pl.when(b0 + 1 < nb)
            def _():
                finish(b0 + 1, 1)

            return carry

        lax.fori_loop(0, (nb + 1) // 2, step, 0)

    return k(pqc, si, di)


def kernel(x, edge_index, params):
    n = x.shape[0]
    npad = ((n + 127) // 128) * 128
    src = edge_index[0]
    dst = edge_index[1]
    srcoff = (src[None, :]
              + (jnp.arange(_FC, dtype=jnp.int32) * npad)[:, None]).reshape(-1)

    h = x
    for p in params['layers']:
        fc = _tc_mlp2_chunked(h, p['f_W1'], p['f_b1'], p['f_W2'], p['f_b2'], npad)
        aggc = _sc_aggregate(fc.reshape(_FC * npad, 128), srcoff, dst, npad)
        h = _tc_out_mlp(fc, aggc.reshape(_FC, npad, 128), p, n)

    w1 = params['lp_W1']
    dh = h.shape[1]
    pq = _tc_dual_mm(h, w1[:dh], w1[dh:])
    ps, qd = _sc_edge_gather(pq.reshape(2 * n, dh), src, dst + n)
    return _tc_edge_mlp(ps, qd, params['lp_b1'], params['lp_W2'], params['lp_b2'])
